# Initial kernel scaffold; baseline (speedup 1.0000x reference)
import jax
import jax.numpy as jnp
from jax.experimental import pallas as pl


def kernel(x, Wg, W1, b1, W2, b2):
    raise NotImplementedError("write your pallas kernel here")


# speedup vs baseline: 2.8292x; 2.8292x over previous
import jax
import jax.numpy as jnp
from jax.experimental import pallas as pl
from jax.experimental.pallas import tpu as pltpu


def _gate_kernel(x_ref, wg_ref, w_ref):
    logits = jax.lax.dot_general(
        x_ref[...], wg_ref[...], (((1,), (1,)), ((), ())),
        preferred_element_type=jnp.float32)
    e_num = logits.shape[1]
    iota = jax.lax.broadcasted_iota(jnp.int32, logits.shape, 1)
    m1 = jnp.max(logits, axis=1, keepdims=True)
    i1 = jnp.min(jnp.where(logits == m1, iota, e_num), axis=1, keepdims=True)
    l2 = jnp.where(iota == i1, -jnp.inf, logits)
    m2 = jnp.max(l2, axis=1, keepdims=True)
    i2 = jnp.min(jnp.where(l2 == m2, iota, e_num), axis=1, keepdims=True)
    z = jnp.exp(m2 - m1)
    wa = 1.0 / (1.0 + z)
    wb = z / (1.0 + z)
    w_ref[...] = jnp.where(iota == i1, wa, 0.0) + jnp.where(iota == i2, wb, 0.0)


def _moe_kernel(w_ref, x_ref, w1_ref, w2_ref, b1_ref, b2_ref, out_ref, *,
                ch: int):
    e = pl.program_id(0)
    n = x_ref.shape[0]
    e_num = w_ref.shape[1]

    def body(r, carry):
        sl = pl.ds(r * ch, ch)
        xb = x_ref[sl, :]
        h = jax.lax.dot_general(
            xb, w1_ref[0], (((1,), (1,)), ((), ())),
            preferred_element_type=jnp.float32)
        h = h + b1_ref[0]
        h = 0.5 * h * (1.0 + jax.lax.erf(h * 0.7071067811865476))
        hb = h.astype(jnp.bfloat16)
        y = jax.lax.dot_general(
            hb, w2_ref[0], (((1,), (1,)), ((), ())),
            preferred_element_type=jnp.float32)
        y = y + b2_ref[0]
        wb = w_ref[sl, :]
        iota = jax.lax.broadcasted_iota(jnp.int32, wb.shape, 1)
        wcol = jnp.sum(jnp.where(iota == e, wb, 0.0), axis=1, keepdims=True)
        contrib = wcol * y

        @pl.when(e == 0)
        def _():
            out_ref[sl, :] = contrib

        @pl.when(e != 0)
        def _():
            out_ref[sl, :] = out_ref[sl, :] + contrib

        return carry

    jax.lax.fori_loop(0, n // ch, body, 0)


def kernel(x, Wg, W1, b1, W2, b2):
    bv, tv, d = x.shape
    n = bv * tv
    e_num, f = W1.shape[0], W1.shape[1]
    xf = x.reshape(n, d)
    xb16 = xf.astype(jnp.bfloat16)
    w1b = W1.astype(jnp.bfloat16)
    w2b = W2.astype(jnp.bfloat16)

    gch = 2048
    w = pl.pallas_call(
        _gate_kernel,
        grid=(n // gch,),
        in_specs=[
            pl.BlockSpec((gch, d), lambda i: (i, 0)),
            pl.BlockSpec((e_num, d), lambda i: (0, 0)),
        ],
        out_specs=pl.BlockSpec((gch, e_num), lambda i: (i, 0)),
        out_shape=jax.ShapeDtypeStruct((n, e_num), jnp.float32),
    )(xf, Wg)

    ch = 512
    import functools
    out = pl.pallas_call(
        functools.partial(_moe_kernel, ch=ch),
        grid=(e_num,),
        in_specs=[
            pl.BlockSpec((n, e_num), lambda e: (0, 0)),
            pl.BlockSpec((n, d), lambda e: (0, 0)),
            pl.BlockSpec((1, f, d), lambda e: (e, 0, 0)),
            pl.BlockSpec((1, d, f), lambda e: (e, 0, 0)),
            pl.BlockSpec((1, 1, f), lambda e: (e, 0, 0)),
            pl.BlockSpec((1, 1, d), lambda e: (e, 0, 0)),
        ],
        out_specs=pl.BlockSpec((n, d), lambda e: (0, 0)),
        out_shape=jax.ShapeDtypeStruct((n, d), jnp.float32),
        compiler_params=pltpu.CompilerParams(
            dimension_semantics=("arbitrary",),
            vmem_limit_bytes=100 * 1024 * 1024,
        ),
    )(w, xb16, w1b, w2b, b1.reshape(e_num, 1, f), b2.reshape(e_num, 1, d))
    return out.reshape(bv, tv, d)
